# baseline (speedup 1.0000x reference)
import functools

import jax
import jax.numpy as jnp
from jax import lax
from jax.experimental import pallas as pl
from jax.experimental.pallas import tpu as pltpu
from jax.experimental.pallas import tpu_sc as plsc

BATCH = 4096
MAXLEN = 200
EMBED = 64
LABELS = 100
LANES = 16
NC = 2
NS = 16
NW = NC * NS
B_PER_W = BATCH // NW
CHUNKS = ((0, 128), (128, 72))


def _pool_body(x_hbm, table_hbm, out_hbm, idx_v, rows_v, pooled_v, sem):
    w = lax.axis_index("s") * NC + lax.axis_index("c")
    base = w * B_PER_W
    pltpu.sync_copy(x_hbm.at[pl.ds(base, B_PER_W)], idx_v)

    def per_row(b, carry):
        copies = [
            pltpu.async_copy(
                table_hbm.at[idx_v.at[b, pl.ds(off, n)]],
                rows_v.at[pl.ds(off, n)],
                sem,
            )
            for off, n in CHUNKS
        ]
        for cp in copies:
            cp.wait()

        def red(r, accs):
            return tuple(
                a + rows_v[r, pl.ds(c * LANES, LANES)]
                for c, a in enumerate(accs)
            )

        accs = lax.fori_loop(
            0, MAXLEN, red,
            tuple(jnp.zeros((LANES,), jnp.float32) for _ in range(EMBED // LANES)),
        )
        scale = jnp.float32(1.0 / MAXLEN)
        for c in range(EMBED // LANES):
            pooled_v[b, pl.ds(c * LANES, LANES)] = accs[c] * scale
        return carry

    lax.fori_loop(0, B_PER_W, per_row, 0)
    pltpu.sync_copy(pooled_v, out_hbm.at[pl.ds(base, B_PER_W)])


_pool_kernel = functools.partial(
    pl.kernel,
    out_type=jax.ShapeDtypeStruct((BATCH, EMBED), jnp.float32),
    mesh=plsc.VectorSubcoreMesh(core_axis_name="c", subcore_axis_name="s"),
    scratch_types=[
        pltpu.VMEM((B_PER_W, MAXLEN), jnp.int32),
        pltpu.VMEM((MAXLEN, EMBED), jnp.float32),
        pltpu.VMEM((B_PER_W, EMBED), jnp.float32),
        pltpu.SemaphoreType.DMA,
    ],
    compiler_params=pltpu.CompilerParams(use_tc_tiling_on_sc=False),
)(_pool_body)


LPAD = 128
BM = 512


def _fc_body(p_ref, w_ref, b_ref, o_ref):
    o_ref[...] = (
        jnp.dot(p_ref[...], w_ref[...], preferred_element_type=jnp.float32)
        + b_ref[0:1, :]
    )


@jax.jit
def kernel(x, table, W, b):
    x = x.astype(jnp.int32)
    pooled = _pool_kernel(x, table)

    wp = jnp.zeros((EMBED, LPAD), jnp.float32).at[:, :LABELS].set(W.T)
    bp = jnp.zeros((8, LPAD), jnp.float32).at[:, :LABELS].set(b[None, :])
    out = pl.pallas_call(
        _fc_body,
        grid=(BATCH // BM,),
        in_specs=[
            pl.BlockSpec((BM, EMBED), lambda i: (i, 0)),
            pl.BlockSpec((EMBED, LPAD), lambda i: (0, 0)),
            pl.BlockSpec((8, LPAD), lambda i: (0, 0)),
        ],
        out_specs=pl.BlockSpec((BM, LPAD), lambda i: (i, 0)),
        out_shape=jax.ShapeDtypeStruct((BATCH, LPAD), jnp.float32),
    )(pooled, wp, bp)
    return out[:, :LABELS]


# speedup vs baseline: 1.2388x; 1.2388x over previous
import functools

import jax
import jax.numpy as jnp
from jax import lax
from jax.experimental import pallas as pl
from jax.experimental.pallas import tpu as pltpu
from jax.experimental.pallas import tpu_sc as plsc

BATCH = 4096
MAXLEN = 200
EMBED = 64
LABELS = 100
LANES = 16
NC = 2
NS = 16
NW = NC * NS
B_PER_W = BATCH // NW
CHUNKS = ((0, 128), (128, 72))


NBUF = 4
UNROLL = 8
NCH = EMBED // LANES


def _pool_body(x_hbm, table_hbm, out_hbm, idx_v, rows_v, pooled_v, sems):
    w = lax.axis_index("s") * NC + lax.axis_index("c")
    base = w * B_PER_W
    pltpu.sync_copy(x_hbm.at[pl.ds(base, B_PER_W)], idx_v)

    def fire(b, k):
        for off, n in CHUNKS:
            pltpu.async_copy(
                table_hbm.at[idx_v.at[b, pl.ds(off, n)]],
                rows_v.at[k].at[pl.ds(off, n)],
                sems.at[k],
            )

    def drain(k):
        pltpu.make_async_copy(
            table_hbm.at[pl.ds(0, MAXLEN)], rows_v.at[k], sems.at[k]
        ).wait()

    def reduce_row(b, k):
        rv = rows_v.at[k]
        zero = jnp.zeros((LANES,), jnp.float32)

        def step(t, accs):
            accs = list(accs)
            for j in range(UNROLL):
                r = t * UNROLL + j
                for c in range(NCH):
                    a = c + NCH * (j % 2)
                    accs[a] = accs[a] + rv[r, pl.ds(c * LANES, LANES)]
            return tuple(accs)

        accs = lax.fori_loop(0, MAXLEN // UNROLL, step, (zero,) * (2 * NCH))
        scale = jnp.float32(1.0 / MAXLEN)
        for c in range(NCH):
            pooled_v[b, pl.ds(c * LANES, LANES)] = (accs[c] + accs[c + NCH]) * scale

    for k in range(NBUF):
        fire(k, k)

    def group(g, carry):
        for k in range(NBUF):
            b = g * NBUF + k
            drain(k)

            @pl.when(g < B_PER_W // NBUF - 1)
            def _():
                fire(b + NBUF, k)

            reduce_row(b, k)
        return carry

    lax.fori_loop(0, B_PER_W // NBUF, group, 0)
    pltpu.sync_copy(pooled_v, out_hbm.at[pl.ds(base, B_PER_W)])


_pool_kernel = functools.partial(
    pl.kernel,
    out_type=jax.ShapeDtypeStruct((BATCH, EMBED), jnp.float32),
    mesh=plsc.VectorSubcoreMesh(core_axis_name="c", subcore_axis_name="s"),
    scratch_types=[
        pltpu.VMEM((B_PER_W, MAXLEN), jnp.int32),
        pltpu.VMEM((NBUF, MAXLEN, EMBED), jnp.float32),
        pltpu.VMEM((B_PER_W, EMBED), jnp.float32),
        pltpu.SemaphoreType.DMA((NBUF,)),
    ],
    compiler_params=pltpu.CompilerParams(use_tc_tiling_on_sc=False),
)(_pool_body)


LPAD = 128
BM = 512


def _fc_body(p_ref, w_ref, b_ref, o_ref):
    o_ref[...] = (
        jnp.dot(p_ref[...], w_ref[...], preferred_element_type=jnp.float32)
        + b_ref[0:1, :]
    )


@jax.jit
def kernel(x, table, W, b):
    x = x.astype(jnp.int32)
    pooled = _pool_kernel(x, table)

    wp = jnp.zeros((EMBED, LPAD), jnp.float32).at[:, :LABELS].set(W.T)
    bp = jnp.zeros((8, LPAD), jnp.float32).at[:, :LABELS].set(b[None, :])
    out = pl.pallas_call(
        _fc_body,
        grid=(BATCH // BM,),
        in_specs=[
            pl.BlockSpec((BM, EMBED), lambda i: (i, 0)),
            pl.BlockSpec((EMBED, LPAD), lambda i: (0, 0)),
            pl.BlockSpec((8, LPAD), lambda i: (0, 0)),
        ],
        out_specs=pl.BlockSpec((BM, LPAD), lambda i: (i, 0)),
        out_shape=jax.ShapeDtypeStruct((BATCH, LPAD), jnp.float32),
    )(pooled, wp, bp)
    return out[:, :LABELS]
